# initial kernel scaffold (unmeasured)
import jax
import jax.numpy as jnp
from jax import lax
from jax.experimental import pallas as pl
from jax.experimental.pallas import tpu as pltpu

N_DEV = 8
M = 1536
N = 1536


def kernel(A, B):
    def body(a_ref, b_ref, out_ref, zacc, s0, s1, s2, r0, r1, r2,
             send_sems, recv_sems):
        my = lax.axis_index("i")
        b0 = my & 1
        b1 = (my >> 1) & 1
        b2 = (my >> 2) & 1
        p_x = my ^ 1
        p_y = my ^ 3
        p_z = my ^ 4

        barrier_sem = pltpu.get_barrier_semaphore()
        for nbr in (p_x, p_y, p_z):
            pl.semaphore_signal(
                barrier_sem, inc=1,
                device_id=(nbr,), device_id_type=pl.DeviceIdType.MESH,
            )
        pl.semaphore_wait(barrier_sem, 3)

        zacc[...] = jnp.dot(
            a_ref[...].astype(jnp.bfloat16),
            b_ref[...].astype(jnp.bfloat16),
            preferred_element_type=jnp.float32,
        )

        seg_off = 0
        for partner, bit, half, sbuf, rbuf, sem in (
            (p_x, b0, 768, s0, r0, 0),
            (p_y, b1, 384, s1, r1, 1),
            (p_z, b2, 192, s2, r2, 2),
        ):
            send_off = seg_off + (1 - bit) * half
            keep_off = seg_off + bit * half
            sbuf[...] = zacc[pl.ds(send_off, half), :].astype(jnp.bfloat16)
            rdma = pltpu.make_async_remote_copy(
                src_ref=sbuf,
                dst_ref=rbuf,
                send_sem=send_sems.at[sem],
                recv_sem=recv_sems.at[sem],
                device_id=(partner,),
                device_id_type=pl.DeviceIdType.MESH,
            )
            rdma.start()
            rdma.wait()
            zacc[pl.ds(keep_off, half), :] = (
                zacc[pl.ds(keep_off, half), :] + rbuf[...].astype(jnp.float32)
            )
            seg_off = keep_off

        z = zacc[pl.ds(seg_off, 192), :]
        out_ref[pl.ds(seg_off, 192), :] = (
            z / (1.0 + jnp.exp(-z))
        ).astype(jnp.bfloat16)

        cur_off = seg_off
        for partner, bit, ln, sem in (
            (p_z, b2, 192, 3),
            (p_y, b1, 384, 4),
            (p_x, b0, 768, 5),
        ):
            rdma = pltpu.make_async_remote_copy(
                src_ref=out_ref.at[pl.ds(cur_off, ln)],
                dst_ref=out_ref.at[pl.ds(cur_off, ln)],
                send_sem=send_sems.at[sem],
                recv_sem=recv_sems.at[sem],
                device_id=(partner,),
                device_id_type=pl.DeviceIdType.MESH,
            )
            rdma.start()
            rdma.wait()
            cur_off = cur_off - bit * ln

    return pl.pallas_call(
        body,
        out_shape=jax.ShapeDtypeStruct((M, N), jnp.bfloat16),
        in_specs=[
            pl.BlockSpec(memory_space=pltpu.VMEM),
            pl.BlockSpec(memory_space=pltpu.VMEM),
        ],
        out_specs=pl.BlockSpec(memory_space=pltpu.VMEM),
        scratch_shapes=[
            pltpu.VMEM((M, N), jnp.float32),
            pltpu.VMEM((768, N), jnp.bfloat16),
            pltpu.VMEM((384, N), jnp.bfloat16),
            pltpu.VMEM((192, N), jnp.bfloat16),
            pltpu.VMEM((768, N), jnp.bfloat16),
            pltpu.VMEM((384, N), jnp.bfloat16),
            pltpu.VMEM((192, N), jnp.bfloat16),
            pltpu.SemaphoreType.DMA((6,)),
            pltpu.SemaphoreType.DMA((6,)),
        ],
        compiler_params=pltpu.CompilerParams(collective_id=0),
    )(A, B)


# baseline (device time: 114846 ns/iter reference)
import jax
import jax.numpy as jnp
from jax import lax
from jax.experimental import pallas as pl
from jax.experimental.pallas import tpu as pltpu

N_DEV = 8
M = 1536
N = 1536


def kernel(A, B):
    def body(a_ref, b_ref, out_ref, zacc, s0, s1, s2, r0, r1, r2,
             send_sems, recv_sems):
        my = lax.axis_index("i")
        cx = (my ^ (my >> 1)) & 1
        cy = (my >> 1) & 1
        cz = (my >> 2) & 1
        p_x = my ^ 1
        p_y = my ^ 3
        p_z = my ^ 4

        barrier_sem = pltpu.get_barrier_semaphore()
        for nbr in (p_x, p_y, p_z):
            pl.semaphore_signal(
                barrier_sem, inc=1,
                device_id=(nbr,), device_id_type=pl.DeviceIdType.MESH,
            )
        pl.semaphore_wait(barrier_sem, 3)

        zacc[...] = jnp.dot(
            a_ref[...].astype(jnp.bfloat16),
            b_ref[...].astype(jnp.bfloat16),
            preferred_element_type=jnp.float32,
        )

        seg_off = 0
        for partner, bit, half, sbuf, rbuf, sem in (
            (p_x, cx, 768, s0, r0, 0),
            (p_y, cy, 384, s1, r1, 1),
            (p_z, cz, 192, s2, r2, 2),
        ):
            send_off = seg_off + (1 - bit) * half
            keep_off = seg_off + bit * half
            sbuf[...] = zacc[pl.ds(send_off, half), :].astype(jnp.bfloat16)
            rdma = pltpu.make_async_remote_copy(
                src_ref=sbuf,
                dst_ref=rbuf,
                send_sem=send_sems.at[sem],
                recv_sem=recv_sems.at[sem],
                device_id=(partner,),
                device_id_type=pl.DeviceIdType.MESH,
            )
            rdma.start()
            rdma.wait()
            zacc[pl.ds(keep_off, half), :] = (
                zacc[pl.ds(keep_off, half), :] + rbuf[...].astype(jnp.float32)
            )
            seg_off = keep_off

        z = zacc[pl.ds(seg_off, 192), :]
        out_ref[pl.ds(seg_off, 192), :] = (
            z / (1.0 + jnp.exp(-z))
        ).astype(jnp.bfloat16)

        cur_off = seg_off
        for partner, bit, ln, sem in (
            (p_z, cz, 192, 3),
            (p_y, cy, 384, 4),
            (p_x, cx, 768, 5),
        ):
            rdma = pltpu.make_async_remote_copy(
                src_ref=out_ref.at[pl.ds(cur_off, ln)],
                dst_ref=out_ref.at[pl.ds(cur_off, ln)],
                send_sem=send_sems.at[sem],
                recv_sem=recv_sems.at[sem],
                device_id=(partner,),
                device_id_type=pl.DeviceIdType.MESH,
            )
            rdma.start()
            rdma.wait()
            cur_off = cur_off - bit * ln

    return pl.pallas_call(
        body,
        out_shape=jax.ShapeDtypeStruct((M, N), jnp.bfloat16),
        in_specs=[
            pl.BlockSpec(memory_space=pltpu.VMEM),
            pl.BlockSpec(memory_space=pltpu.VMEM),
        ],
        out_specs=pl.BlockSpec(memory_space=pltpu.VMEM),
        scratch_shapes=[
            pltpu.VMEM((M, N), jnp.float32),
            pltpu.VMEM((768, N), jnp.bfloat16),
            pltpu.VMEM((384, N), jnp.bfloat16),
            pltpu.VMEM((192, N), jnp.bfloat16),
            pltpu.VMEM((768, N), jnp.bfloat16),
            pltpu.VMEM((384, N), jnp.bfloat16),
            pltpu.VMEM((192, N), jnp.bfloat16),
            pltpu.SemaphoreType.DMA((6,)),
            pltpu.SemaphoreType.DMA((6,)),
        ],
        compiler_params=pltpu.CompilerParams(collective_id=0),
    )(A, B)


# device time: 55268 ns/iter; 2.0780x vs baseline; 2.0780x over previous
import jax
import jax.numpy as jnp
from jax import lax
from jax.experimental import pallas as pl
from jax.experimental.pallas import tpu as pltpu

N_DEV = 8
M = 1536
N = 1536
P_ROWS = M // 3


def kernel(A, B):
    def body(a_ref, b_ref, out_ref, zacc,
             s0, s1, s2, s3, s4, s5, s6, s7, s8,
             r0, r1, r2, r3, r4, r5, r6, r7, r8,
             send_sems, recv_sems):
        sbufs = ((s0, s1, s2), (s3, s4, s5), (s6, s7, s8))
        rbufs = ((r0, r1, r2), (r3, r4, r5), (r6, r7, r8))

        my = lax.axis_index("i")
        cx = (my ^ (my >> 1)) & 1
        cy = (my >> 1) & 1
        cz = (my >> 2) & 1
        ax_x = (my ^ 1, cx)
        ax_y = (my ^ 3, cy)
        ax_z = (my ^ 4, cz)
        orders = ((ax_x, ax_y, ax_z), (ax_y, ax_z, ax_x), (ax_z, ax_x, ax_y))

        barrier_sem = pltpu.get_barrier_semaphore()
        for nbr, _ in (ax_x, ax_y, ax_z):
            pl.semaphore_signal(
                barrier_sem, inc=1,
                device_id=(nbr,), device_id_type=pl.DeviceIdType.MESH,
            )
        pl.semaphore_wait(barrier_sem, 3)

        zacc[...] = jnp.dot(
            a_ref[...].astype(jnp.bfloat16),
            b_ref[...].astype(jnp.bfloat16),
            preferred_element_type=jnp.float32,
        )

        seg = [0 * my + P_ROWS * p for p in range(3)]
        for k in range(3):
            rows = 256 >> k
            rdmas = []
            for p in range(3):
                partner, bit = orders[p][k]
                send_off = seg[p] + (1 - bit) * rows
                sbufs[p][k][...] = (
                    zacc[pl.ds(send_off, rows), :].astype(jnp.bfloat16)
                )
                rdma = pltpu.make_async_remote_copy(
                    src_ref=sbufs[p][k],
                    dst_ref=rbufs[p][k],
                    send_sem=send_sems.at[3 * k + p],
                    recv_sem=recv_sems.at[3 * k + p],
                    device_id=(partner,),
                    device_id_type=pl.DeviceIdType.MESH,
                )
                rdma.start()
                rdmas.append(rdma)
            for p in range(3):
                rdmas[p].wait()
                _, bit = orders[p][k]
                keep_off = seg[p] + bit * rows
                zacc[pl.ds(keep_off, rows), :] = (
                    zacc[pl.ds(keep_off, rows), :]
                    + rbufs[p][k][...].astype(jnp.float32)
                )
                seg[p] = keep_off

        for p in range(3):
            z = zacc[pl.ds(seg[p], 64), :]
            out_ref[pl.ds(seg[p], 64), :] = (
                z / (1.0 + jnp.exp(-z))
            ).astype(jnp.bfloat16)

        cur = list(seg)
        for k in range(3):
            ln = 64 << k
            rdmas = []
            for p in range(3):
                partner, bit = orders[p][2 - k]
                rdma = pltpu.make_async_remote_copy(
                    src_ref=out_ref.at[pl.ds(cur[p], ln)],
                    dst_ref=out_ref.at[pl.ds(cur[p], ln)],
                    send_sem=send_sems.at[9 + 3 * k + p],
                    recv_sem=recv_sems.at[9 + 3 * k + p],
                    device_id=(partner,),
                    device_id_type=pl.DeviceIdType.MESH,
                )
                rdma.start()
                rdmas.append(rdma)
            for p in range(3):
                rdmas[p].wait()
                _, bit = orders[p][2 - k]
                cur[p] = cur[p] - bit * ln

    return pl.pallas_call(
        body,
        out_shape=jax.ShapeDtypeStruct((M, N), jnp.bfloat16),
        in_specs=[
            pl.BlockSpec(memory_space=pltpu.VMEM),
            pl.BlockSpec(memory_space=pltpu.VMEM),
        ],
        out_specs=pl.BlockSpec(memory_space=pltpu.VMEM),
        scratch_shapes=[
            pltpu.VMEM((M, N), jnp.float32),
            pltpu.VMEM((256, N), jnp.bfloat16),
            pltpu.VMEM((128, N), jnp.bfloat16),
            pltpu.VMEM((64, N), jnp.bfloat16),
            pltpu.VMEM((256, N), jnp.bfloat16),
            pltpu.VMEM((128, N), jnp.bfloat16),
            pltpu.VMEM((64, N), jnp.bfloat16),
            pltpu.VMEM((256, N), jnp.bfloat16),
            pltpu.VMEM((128, N), jnp.bfloat16),
            pltpu.VMEM((64, N), jnp.bfloat16),
            pltpu.VMEM((256, N), jnp.bfloat16),
            pltpu.VMEM((128, N), jnp.bfloat16),
            pltpu.VMEM((64, N), jnp.bfloat16),
            pltpu.VMEM((256, N), jnp.bfloat16),
            pltpu.VMEM((128, N), jnp.bfloat16),
            pltpu.VMEM((64, N), jnp.bfloat16),
            pltpu.VMEM((256, N), jnp.bfloat16),
            pltpu.VMEM((128, N), jnp.bfloat16),
            pltpu.VMEM((64, N), jnp.bfloat16),
            pltpu.SemaphoreType.DMA((18,)),
            pltpu.SemaphoreType.DMA((18,)),
        ],
        compiler_params=pltpu.CompilerParams(collective_id=0),
    )(A, B)


# device time: 53553 ns/iter; 2.1445x vs baseline; 1.0320x over previous
import jax
import jax.numpy as jnp
from jax import lax
from jax.experimental import pallas as pl
from jax.experimental.pallas import tpu as pltpu

N_DEV = 8
M = 1536
N = 1536
P_ROWS = M // 3


def kernel(A, B):
    def body(a_ref, b_ref, out_ref, zacc,
             s0, s1, s2, s3, s4, s5, s6, s7, s8,
             r0, r1, r2, r3, r4, r5, r6, r7, r8,
             send_sems, recv_sems):
        sbufs = ((s0, s1, s2), (s3, s4, s5), (s6, s7, s8))
        rbufs = ((r0, r1, r2), (r3, r4, r5), (r6, r7, r8))

        my = lax.axis_index("i")
        cx = (my ^ (my >> 1)) & 1
        cy = (my >> 1) & 1
        cz = (my >> 2) & 1
        ax_x = (my ^ 1, cx)
        ax_y = (my ^ 3, cy)
        ax_z = (my ^ 4, cz)
        orders = ((ax_x, ax_y, ax_z), (ax_y, ax_z, ax_x), (ax_z, ax_x, ax_y))

        bf16 = jnp.bfloat16
        f32 = jnp.float32

        def rs_rdma(p, k, partner):
            return pltpu.make_async_remote_copy(
                src_ref=sbufs[p][k],
                dst_ref=rbufs[p][k],
                send_sem=send_sems.at[3 * k + p],
                recv_sem=recv_sems.at[3 * k + p],
                device_id=(partner,),
                device_id_type=pl.DeviceIdType.MESH,
            )

        barrier_sem = pltpu.get_barrier_semaphore()
        for nbr, _ in (ax_x, ax_y, ax_z):
            pl.semaphore_signal(
                barrier_sem, inc=1,
                device_id=(nbr,), device_id_type=pl.DeviceIdType.MESH,
            )
        pl.semaphore_wait(barrier_sem, 3)

        b_bf = b_ref[...].astype(bf16)

        rdmas = []
        keep0 = []
        for p in range(3):
            partner, bit = orders[p][0]
            send_off = P_ROWS * p + (1 - bit) * 256
            keep0.append(P_ROWS * p + bit * 256)
            sbufs[p][0][...] = jnp.dot(
                a_ref[pl.ds(send_off, 256), :].astype(bf16), b_bf,
                preferred_element_type=f32,
            ).astype(bf16)
            rdma = rs_rdma(p, 0, partner)
            rdma.start()
            rdmas.append(rdma)

        for p in range(3):
            zacc[pl.ds(keep0[p], 256), :] = jnp.dot(
                a_ref[pl.ds(keep0[p], 256), :].astype(bf16), b_bf,
                preferred_element_type=f32,
            )

        seg = keep0
        for k in (1, 2):
            rows = 256 >> k
            nxt = []
            for p in range(3):
                rdmas[p].wait()
                partner, bit = orders[p][k]
                sbufs[p][k][...] = (
                    zacc[pl.ds(seg[p] + (1 - bit) * rows, rows), :]
                    + rbufs[p][k - 1][pl.ds((1 - bit) * rows, rows), :]
                    .astype(f32)
                ).astype(bf16)
                rdma = rs_rdma(p, k, partner)
                rdma.start()
                nxt.append(rdma)
            for p in range(3):
                _, bit = orders[p][k]
                keep_off = seg[p] + bit * rows
                zacc[pl.ds(keep_off, rows), :] = (
                    zacc[pl.ds(keep_off, rows), :]
                    + rbufs[p][k - 1][pl.ds(bit * rows, rows), :].astype(f32)
                )
                seg[p] = keep_off
            rdmas = nxt

        for p in range(3):
            rdmas[p].wait()
            z = (
                zacc[pl.ds(seg[p], 64), :]
                + rbufs[p][2][...].astype(f32)
            )
            out_ref[pl.ds(seg[p], 64), :] = (
                z / (1.0 + jnp.exp(-z))
            ).astype(bf16)

        cur = list(seg)
        for k in range(3):
            ln = 64 << k
            rdmas = []
            for p in range(3):
                partner, bit = orders[p][2 - k]
                rdma = pltpu.make_async_remote_copy(
                    src_ref=out_ref.at[pl.ds(cur[p], ln)],
                    dst_ref=out_ref.at[pl.ds(cur[p], ln)],
                    send_sem=send_sems.at[9 + 3 * k + p],
                    recv_sem=recv_sems.at[9 + 3 * k + p],
                    device_id=(partner,),
                    device_id_type=pl.DeviceIdType.MESH,
                )
                rdma.start()
                rdmas.append(rdma)
            for p in range(3):
                rdmas[p].wait()
                _, bit = orders[p][2 - k]
                cur[p] = cur[p] - bit * ln

    return pl.pallas_call(
        body,
        out_shape=jax.ShapeDtypeStruct((M, N), jnp.bfloat16),
        in_specs=[
            pl.BlockSpec(memory_space=pltpu.VMEM),
            pl.BlockSpec(memory_space=pltpu.VMEM),
        ],
        out_specs=pl.BlockSpec(memory_space=pltpu.VMEM),
        scratch_shapes=[
            pltpu.VMEM((M, N), jnp.float32),
            pltpu.VMEM((256, N), jnp.bfloat16),
            pltpu.VMEM((128, N), jnp.bfloat16),
            pltpu.VMEM((64, N), jnp.bfloat16),
            pltpu.VMEM((256, N), jnp.bfloat16),
            pltpu.VMEM((128, N), jnp.bfloat16),
            pltpu.VMEM((64, N), jnp.bfloat16),
            pltpu.VMEM((256, N), jnp.bfloat16),
            pltpu.VMEM((128, N), jnp.bfloat16),
            pltpu.VMEM((64, N), jnp.bfloat16),
            pltpu.VMEM((256, N), jnp.bfloat16),
            pltpu.VMEM((128, N), jnp.bfloat16),
            pltpu.VMEM((64, N), jnp.bfloat16),
            pltpu.VMEM((256, N), jnp.bfloat16),
            pltpu.VMEM((128, N), jnp.bfloat16),
            pltpu.VMEM((64, N), jnp.bfloat16),
            pltpu.VMEM((256, N), jnp.bfloat16),
            pltpu.VMEM((128, N), jnp.bfloat16),
            pltpu.VMEM((64, N), jnp.bfloat16),
            pltpu.SemaphoreType.DMA((18,)),
            pltpu.SemaphoreType.DMA((18,)),
        ],
        compiler_params=pltpu.CompilerParams(collective_id=0),
    )(A, B)


# device time: 45869 ns/iter; 2.5038x vs baseline; 1.1675x over previous
import jax
import jax.numpy as jnp
from jax import lax
from jax.experimental import pallas as pl
from jax.experimental.pallas import tpu as pltpu

N_DEV = 8
M = 1536
N = 1536
P_ROWS = M // 3
C = 2
CN = N // C
RS_ROWS = (256, 128, 64)
AG_ROWS = (64, 128, 256)


def kernel(A, B):
    def body(a_ref, b_ref, out_ref, zacc, *rest):
        flat_s, flat_r = rest[:18], rest[18:36]
        send_sems, recv_sems = rest[36], rest[37]
        sbufs = [[[flat_s[(c * 3 + p) * 3 + k] for k in range(3)]
                  for p in range(3)] for c in range(C)]
        rbufs = [[[flat_r[(c * 3 + p) * 3 + k] for k in range(3)]
                  for p in range(3)] for c in range(C)]

        my = lax.axis_index("i")
        cx = (my ^ (my >> 1)) & 1
        cy = (my >> 1) & 1
        cz = (my >> 2) & 1
        ax_x = (my ^ 1, cx)
        ax_y = (my ^ 3, cy)
        ax_z = (my ^ 4, cz)
        orders = ((ax_x, ax_y, ax_z), (ax_y, ax_z, ax_x), (ax_z, ax_x, ax_y))

        bf16 = jnp.bfloat16
        f32 = jnp.float32

        def cslice(c):
            return pl.ds(c * CN, CN)

        def rs_rdma(c, p, k):
            partner, _ = orders[p][k]
            return pltpu.make_async_remote_copy(
                src_ref=sbufs[c][p][k],
                dst_ref=rbufs[c][p][k],
                send_sem=send_sems.at[c * 18 + 3 * k + p],
                recv_sem=recv_sems.at[c * 18 + 3 * k + p],
                device_id=(partner,),
                device_id_type=pl.DeviceIdType.MESH,
            )

        def ag_rdma(c, p, k, off):
            partner, _ = orders[p][2 - k]
            return pltpu.make_async_remote_copy(
                src_ref=out_ref.at[pl.ds(off, AG_ROWS[k]), cslice(c)],
                dst_ref=out_ref.at[pl.ds(off, AG_ROWS[k]), cslice(c)],
                send_sem=send_sems.at[c * 18 + 9 + 3 * k + p],
                recv_sem=recv_sems.at[c * 18 + 9 + 3 * k + p],
                device_id=(partner,),
                device_id_type=pl.DeviceIdType.MESH,
            )

        barrier_sem = pltpu.get_barrier_semaphore()
        for nbr, _ in (ax_x, ax_y, ax_z):
            pl.semaphore_signal(
                barrier_sem, inc=1,
                device_id=(nbr,), device_id_type=pl.DeviceIdType.MESH,
            )
        pl.semaphore_wait(barrier_sem, 3)

        b_bf = [b_ref[:, c * CN:(c + 1) * CN].astype(bf16) for c in range(C)]

        rdmas = [[None] * 3 for _ in range(C)]
        seg = [[0] * 3 for _ in range(C)]

        def send_dots(c):
            for p in range(3):
                _, bit = orders[p][0]
                send_off = P_ROWS * p + (1 - bit) * 256
                seg[c][p] = P_ROWS * p + bit * 256
                sbufs[c][p][0][...] = jnp.dot(
                    a_ref[pl.ds(send_off, 256), :].astype(bf16),
                    b_bf[c],
                    preferred_element_type=f32,
                ).astype(bf16)
                rdmas[c][p] = rs_rdma(c, p, 0)
                rdmas[c][p].start()

        def keep_dots(c):
            for p in range(3):
                zacc[pl.ds(seg[c][p], 256), cslice(c)] = jnp.dot(
                    a_ref[pl.ds(seg[c][p], 256), :].astype(bf16),
                    b_bf[c],
                    preferred_element_type=f32,
                )

        def rs_step(c, k):
            rows = RS_ROWS[k]
            for p in range(3):
                rdmas[c][p].wait()
                _, bit = orders[p][k]
                sbufs[c][p][k][...] = (
                    zacc[pl.ds(seg[c][p] + (1 - bit) * rows, rows), cslice(c)]
                    + rbufs[c][p][k - 1][pl.ds((1 - bit) * rows, rows), :]
                    .astype(f32)
                ).astype(bf16)
                rdmas[c][p] = rs_rdma(c, p, k)
                rdmas[c][p].start()
            for p in range(3):
                _, bit = orders[p][k]
                keep_off = seg[c][p] + bit * rows
                zacc[pl.ds(keep_off, rows), cslice(c)] = (
                    zacc[pl.ds(keep_off, rows), cslice(c)]
                    + rbufs[c][p][k - 1][pl.ds(bit * rows, rows), :]
                    .astype(f32)
                )
                seg[c][p] = keep_off

        def rs_final(c):
            for p in range(3):
                rdmas[c][p].wait()
                z = (
                    zacc[pl.ds(seg[c][p], 64), cslice(c)]
                    + rbufs[c][p][2][...].astype(f32)
                )
                out_ref[pl.ds(seg[c][p], 64), cslice(c)] = (
                    z / (1.0 + jnp.exp(-z))
                ).astype(bf16)
                rdmas[c][p] = ag_rdma(c, p, 0, seg[c][p])
                rdmas[c][p].start()

        def ag_step(c, k):
            for p in range(3):
                rdmas[c][p].wait()
                _, bit = orders[p][2 - (k - 1)]
                seg[c][p] = seg[c][p] - bit * AG_ROWS[k - 1]
                rdmas[c][p] = ag_rdma(c, p, k, seg[c][p])
                rdmas[c][p].start()

        def ag_final(c):
            for p in range(3):
                rdmas[c][p].wait()

        send_dots(0)
        send_dots(1)
        keep_dots(0)
        keep_dots(1)
        rs_step(0, 1)
        rs_step(1, 1)
        rs_step(0, 2)
        rs_step(1, 2)
        rs_final(0)
        rs_final(1)
        ag_step(0, 1)
        ag_step(1, 1)
        ag_step(0, 2)
        ag_step(1, 2)
        ag_final(0)
        ag_final(1)

    scratch = [pltpu.VMEM((M, N), jnp.float32)]
    for _ in range(C * 3):
        for rows in RS_ROWS:
            scratch.append(pltpu.VMEM((rows, CN), jnp.bfloat16))
    for _ in range(C * 3):
        for rows in RS_ROWS:
            scratch.append(pltpu.VMEM((rows, CN), jnp.bfloat16))
    scratch.append(pltpu.SemaphoreType.DMA((C * 18,)))
    scratch.append(pltpu.SemaphoreType.DMA((C * 18,)))

    return pl.pallas_call(
        body,
        out_shape=jax.ShapeDtypeStruct((M, N), jnp.bfloat16),
        in_specs=[
            pl.BlockSpec(memory_space=pltpu.VMEM),
            pl.BlockSpec(memory_space=pltpu.VMEM),
        ],
        out_specs=pl.BlockSpec(memory_space=pltpu.VMEM),
        scratch_shapes=scratch,
        compiler_params=pltpu.CompilerParams(collective_id=0),
    )(A, B)


# device time: 44905 ns/iter; 2.5575x vs baseline; 1.0215x over previous
import jax
import jax.numpy as jnp
from jax import lax
from jax.experimental import pallas as pl
from jax.experimental.pallas import tpu as pltpu

N_DEV = 8
M = 1536
N = 1536
P_ROWS = M // 3
C = 3
CN = N // C
RS_ROWS = (256, 128, 64)
AG_ROWS = (64, 128, 256)


def kernel(A, B):
    def body(a_ref, b_ref, out_ref, zacc, *rest):
        nbuf = C * 9
        flat_s, flat_r = rest[:nbuf], rest[nbuf:2 * nbuf]
        send_sems, recv_sems = rest[2 * nbuf], rest[2 * nbuf + 1]
        sbufs = [[[flat_s[(c * 3 + p) * 3 + k] for k in range(3)]
                  for p in range(3)] for c in range(C)]
        rbufs = [[[flat_r[(c * 3 + p) * 3 + k] for k in range(3)]
                  for p in range(3)] for c in range(C)]

        my = lax.axis_index("i")
        cx = (my ^ (my >> 1)) & 1
        cy = (my >> 1) & 1
        cz = (my >> 2) & 1
        ax_x = (my ^ 1, cx)
        ax_y = (my ^ 3, cy)
        ax_z = (my ^ 4, cz)
        orders = ((ax_x, ax_y, ax_z), (ax_y, ax_z, ax_x), (ax_z, ax_x, ax_y))

        bf16 = jnp.bfloat16
        f32 = jnp.float32

        def cslice(c):
            return pl.ds(c * CN, CN)

        def rs_rdma(c, p, k):
            partner, _ = orders[p][k]
            return pltpu.make_async_remote_copy(
                src_ref=sbufs[c][p][k],
                dst_ref=rbufs[c][p][k],
                send_sem=send_sems.at[c * 18 + 3 * k + p],
                recv_sem=recv_sems.at[c * 18 + 3 * k + p],
                device_id=(partner,),
                device_id_type=pl.DeviceIdType.MESH,
            )

        def ag_rdma(c, p, k, off):
            partner, _ = orders[p][2 - k]
            return pltpu.make_async_remote_copy(
                src_ref=out_ref.at[pl.ds(off, AG_ROWS[k]), cslice(c)],
                dst_ref=out_ref.at[pl.ds(off, AG_ROWS[k]), cslice(c)],
                send_sem=send_sems.at[c * 18 + 9 + 3 * k + p],
                recv_sem=recv_sems.at[c * 18 + 9 + 3 * k + p],
                device_id=(partner,),
                device_id_type=pl.DeviceIdType.MESH,
            )

        barrier_sem = pltpu.get_barrier_semaphore()
        for nbr, _ in (ax_x, ax_y, ax_z):
            pl.semaphore_signal(
                barrier_sem, inc=1,
                device_id=(nbr,), device_id_type=pl.DeviceIdType.MESH,
            )
        pl.semaphore_wait(barrier_sem, 3)

        b_bf = [b_ref[:, c * CN:(c + 1) * CN].astype(bf16) for c in range(C)]

        rdmas = [[None] * 3 for _ in range(C)]
        seg = [[0] * 3 for _ in range(C)]

        def send_dots(c):
            for p in range(3):
                _, bit = orders[p][0]
                send_off = P_ROWS * p + (1 - bit) * 256
                seg[c][p] = P_ROWS * p + bit * 256
                sbufs[c][p][0][...] = jnp.dot(
                    a_ref[pl.ds(send_off, 256), :].astype(bf16),
                    b_bf[c],
                    preferred_element_type=f32,
                ).astype(bf16)
                rdmas[c][p] = rs_rdma(c, p, 0)
                rdmas[c][p].start()

        def keep_dots(c):
            for p in range(3):
                zacc[pl.ds(seg[c][p], 256), cslice(c)] = jnp.dot(
                    a_ref[pl.ds(seg[c][p], 256), :].astype(bf16),
                    b_bf[c],
                    preferred_element_type=f32,
                )

        def rs_step(c, k):
            rows = RS_ROWS[k]
            for p in range(3):
                rdmas[c][p].wait()
                _, bit = orders[p][k]
                sbufs[c][p][k][...] = (
                    zacc[pl.ds(seg[c][p] + (1 - bit) * rows, rows), cslice(c)]
                    + rbufs[c][p][k - 1][pl.ds((1 - bit) * rows, rows), :]
                    .astype(f32)
                ).astype(bf16)
                rdmas[c][p] = rs_rdma(c, p, k)
                rdmas[c][p].start()
            for p in range(3):
                _, bit = orders[p][k]
                keep_off = seg[c][p] + bit * rows
                zacc[pl.ds(keep_off, rows), cslice(c)] = (
                    zacc[pl.ds(keep_off, rows), cslice(c)]
                    + rbufs[c][p][k - 1][pl.ds(bit * rows, rows), :]
                    .astype(f32)
                )
                seg[c][p] = keep_off

        def rs_final(c):
            for p in range(3):
                rdmas[c][p].wait()
                z = (
                    zacc[pl.ds(seg[c][p], 64), cslice(c)]
                    + rbufs[c][p][2][...].astype(f32)
                )
                out_ref[pl.ds(seg[c][p], 64), cslice(c)] = (
                    z / (1.0 + jnp.exp(-z))
                ).astype(bf16)
                rdmas[c][p] = ag_rdma(c, p, 0, seg[c][p])
                rdmas[c][p].start()

        def ag_step(c, k):
            for p in range(3):
                rdmas[c][p].wait()
                _, bit = orders[p][2 - (k - 1)]
                seg[c][p] = seg[c][p] - bit * AG_ROWS[k - 1]
                rdmas[c][p] = ag_rdma(c, p, k, seg[c][p])
                rdmas[c][p].start()

        def ag_final(c):
            for p in range(3):
                rdmas[c][p].wait()

        steps = (
            send_dots,
            keep_dots,
            lambda c: rs_step(c, 1),
            lambda c: rs_step(c, 2),
            rs_final,
            lambda c: ag_step(c, 1),
            lambda c: ag_step(c, 2),
            ag_final,
        )
        for step in steps:
            for c in range(C):
                step(c)

    scratch = [pltpu.VMEM((M, N), jnp.float32)]
    for _ in range(C * 3):
        for rows in RS_ROWS:
            scratch.append(pltpu.VMEM((rows, CN), jnp.bfloat16))
    for _ in range(C * 3):
        for rows in RS_ROWS:
            scratch.append(pltpu.VMEM((rows, CN), jnp.bfloat16))
    scratch.append(pltpu.SemaphoreType.DMA((C * 18,)))
    scratch.append(pltpu.SemaphoreType.DMA((C * 18,)))

    return pl.pallas_call(
        body,
        out_shape=jax.ShapeDtypeStruct((M, N), jnp.bfloat16),
        in_specs=[
            pl.BlockSpec(memory_space=pltpu.VMEM),
            pl.BlockSpec(memory_space=pltpu.VMEM),
        ],
        out_specs=pl.BlockSpec(memory_space=pltpu.VMEM),
        scratch_shapes=scratch,
        compiler_params=pltpu.CompilerParams(collective_id=0),
    )(A, B)


# device time: 43838 ns/iter; 2.6198x vs baseline; 1.0243x over previous
import jax
import jax.numpy as jnp
from jax import lax
from jax.experimental import pallas as pl
from jax.experimental.pallas import tpu as pltpu

N_DEV = 8
M = 1536
N = 1536
P_ROWS = M // 3
C = 3
CN = N // C
BUF_ROWS = (256, 128, 128)
SEMS_PER_CHUNK = 15


def kernel(A, B):
    def body(a_ref, b_ref, out_ref, zacc, *rest):
        nbuf = C * 9
        flat_s, flat_r = rest[:nbuf], rest[nbuf:2 * nbuf]
        send_sems, recv_sems = rest[2 * nbuf], rest[2 * nbuf + 1]
        sbufs = [[[flat_s[(c * 3 + p) * 3 + k] for k in range(3)]
                  for p in range(3)] for c in range(C)]
        rbufs = [[[flat_r[(c * 3 + p) * 3 + k] for k in range(3)]
                  for p in range(3)] for c in range(C)]

        my = lax.axis_index("i")
        cx = (my ^ (my >> 1)) & 1
        cy = (my >> 1) & 1
        cz = (my >> 2) & 1
        ax_x = (my ^ 1, cx)
        ax_y = (my ^ 3, cy)
        ax_z = (my ^ 4, cz)
        orders = ((ax_x, ax_y, ax_z), (ax_y, ax_z, ax_x), (ax_z, ax_x, ax_y))

        bf16 = jnp.bfloat16
        f32 = jnp.float32

        def cslice(c):
            return pl.ds(c * CN, CN)

        def rs_rdma(c, p, k):
            partner, _ = orders[p][k]
            return pltpu.make_async_remote_copy(
                src_ref=sbufs[c][p][k],
                dst_ref=rbufs[c][p][k],
                send_sem=send_sems.at[c * SEMS_PER_CHUNK + 3 * k + p],
                recv_sem=recv_sems.at[c * SEMS_PER_CHUNK + 3 * k + p],
                device_id=(partner,),
                device_id_type=pl.DeviceIdType.MESH,
            )

        def ag_rdma(c, p, k, off, rows):
            partner, _ = orders[p][1 - k]
            return pltpu.make_async_remote_copy(
                src_ref=out_ref.at[pl.ds(off, rows), cslice(c)],
                dst_ref=out_ref.at[pl.ds(off, rows), cslice(c)],
                send_sem=send_sems.at[c * SEMS_PER_CHUNK + 9 + 3 * k + p],
                recv_sem=recv_sems.at[c * SEMS_PER_CHUNK + 9 + 3 * k + p],
                device_id=(partner,),
                device_id_type=pl.DeviceIdType.MESH,
            )

        barrier_sem = pltpu.get_barrier_semaphore()
        for nbr, _ in (ax_x, ax_y, ax_z):
            pl.semaphore_signal(
                barrier_sem, inc=1,
                device_id=(nbr,), device_id_type=pl.DeviceIdType.MESH,
            )
        pl.semaphore_wait(barrier_sem, 3)

        b_bf = [b_ref[:, c * CN:(c + 1) * CN].astype(bf16) for c in range(C)]

        rdmas = [[None] * 3 for _ in range(C)]
        seg = [[0] * 3 for _ in range(C)]

        def send_dots(c):
            for p in range(3):
                _, bit = orders[p][0]
                send_off = P_ROWS * p + (1 - bit) * 256
                seg[c][p] = P_ROWS * p + bit * 256
                sbufs[c][p][0][...] = jnp.dot(
                    a_ref[pl.ds(send_off, 256), :].astype(bf16),
                    b_bf[c],
                    preferred_element_type=f32,
                ).astype(bf16)
                rdmas[c][p] = rs_rdma(c, p, 0)
                rdmas[c][p].start()

        def keep_dots(c):
            for p in range(3):
                zacc[pl.ds(seg[c][p], 256), cslice(c)] = jnp.dot(
                    a_ref[pl.ds(seg[c][p], 256), :].astype(bf16),
                    b_bf[c],
                    preferred_element_type=f32,
                )

        def rs_step(c):
            for p in range(3):
                rdmas[c][p].wait()
                _, bit = orders[p][1]
                sbufs[c][p][1][...] = (
                    zacc[pl.ds(seg[c][p] + (1 - bit) * 128, 128), cslice(c)]
                    + rbufs[c][p][0][pl.ds((1 - bit) * 128, 128), :]
                    .astype(f32)
                ).astype(bf16)
                rdmas[c][p] = rs_rdma(c, p, 1)
                rdmas[c][p].start()
            for p in range(3):
                _, bit = orders[p][1]
                keep_off = seg[c][p] + bit * 128
                zacc[pl.ds(keep_off, 128), cslice(c)] = (
                    zacc[pl.ds(keep_off, 128), cslice(c)]
                    + rbufs[c][p][0][pl.ds(bit * 128, 128), :].astype(f32)
                )
                seg[c][p] = keep_off

        def merge_send(c):
            for p in range(3):
                rdmas[c][p].wait()
                sbufs[c][p][2][...] = (
                    zacc[pl.ds(seg[c][p], 128), cslice(c)]
                    + rbufs[c][p][1][...].astype(f32)
                ).astype(bf16)
                rdmas[c][p] = rs_rdma(c, p, 2)
                rdmas[c][p].start()
            for p in range(3):
                zacc[pl.ds(seg[c][p], 128), cslice(c)] = (
                    zacc[pl.ds(seg[c][p], 128), cslice(c)]
                    + rbufs[c][p][1][...].astype(f32)
                )

        def merge_silu(c):
            for p in range(3):
                rdmas[c][p].wait()
                z = (
                    zacc[pl.ds(seg[c][p], 128), cslice(c)]
                    + rbufs[c][p][2][...].astype(f32)
                )
                out_ref[pl.ds(seg[c][p], 128), cslice(c)] = (
                    z / (1.0 + jnp.exp(-z))
                ).astype(bf16)
                rdmas[c][p] = ag_rdma(c, p, 0, seg[c][p], 128)
                rdmas[c][p].start()

        def ag_step(c):
            for p in range(3):
                rdmas[c][p].wait()
                _, bit = orders[p][1]
                seg[c][p] = seg[c][p] - bit * 128
                rdmas[c][p] = ag_rdma(c, p, 1, seg[c][p], 256)
                rdmas[c][p].start()

        def ag_final(c):
            for p in range(3):
                rdmas[c][p].wait()

        steps = (
            send_dots,
            keep_dots,
            rs_step,
            merge_send,
            merge_silu,
            ag_step,
            ag_final,
        )
        for step in steps:
            for c in range(C):
                step(c)

    scratch = [pltpu.VMEM((M, N), jnp.float32)]
    for _ in range(C * 3):
        for rows in BUF_ROWS:
            scratch.append(pltpu.VMEM((rows, CN), jnp.bfloat16))
    for _ in range(C * 3):
        for rows in BUF_ROWS:
            scratch.append(pltpu.VMEM((rows, CN), jnp.bfloat16))
    scratch.append(pltpu.SemaphoreType.DMA((C * SEMS_PER_CHUNK,)))
    scratch.append(pltpu.SemaphoreType.DMA((C * SEMS_PER_CHUNK,)))

    return pl.pallas_call(
        body,
        out_shape=jax.ShapeDtypeStruct((M, N), jnp.bfloat16),
        in_specs=[
            pl.BlockSpec(memory_space=pltpu.VMEM),
            pl.BlockSpec(memory_space=pltpu.VMEM),
        ],
        out_specs=pl.BlockSpec(memory_space=pltpu.VMEM),
        scratch_shapes=scratch,
        compiler_params=pltpu.CompilerParams(collective_id=0),
    )(A, B)
